# Initial kernel scaffold; baseline (speedup 1.0000x reference)
import jax
import jax.numpy as jnp
from jax.experimental import pallas as pl


def kernel(x, weights, ensembles, kernels_0, kernels_1):
    raise NotImplementedError("write your pallas kernel here")


# speedup vs baseline: 6.4207x; 6.4207x over previous
import jax
import jax.numpy as jnp
from jax.experimental import pallas as pl
from jax.experimental.pallas import tpu as pltpu


def _ffe_body(ens_ref, w_ref, x_ref, k0_ref, k1_ref, o_ref):
    e = pl.program_id(0)
    c = jnp.sum(jnp.where(ens_ref[...] == e, w_ref[...], 0.0), axis=1,
                keepdims=True)
    h = jax.lax.dot_general(x_ref[...], k0_ref[0],
                            (((1,), (1,)), ((), ())),
                            preferred_element_type=jnp.float32)
    h = jnp.maximum(h, 0.0) * c
    y = jax.lax.dot_general(h, k1_ref[0],
                            (((1,), (1,)), ((), ())),
                            preferred_element_type=jnp.float32)

    @pl.when(e == 0)
    def _():
        o_ref[...] = jnp.zeros_like(o_ref)

    o_ref[...] += y


def kernel(x, weights, ensembles, kernels_0, kernels_1):
    B, S, D = x.shape
    E, BK, _ = kernels_0.shape
    _, K, _ = weights.shape
    T = B * S

    x2 = x.reshape(T, D)
    ens2 = ensembles.transpose(0, 2, 1).reshape(T, K).astype(jnp.int32)
    w2 = weights.transpose(0, 2, 1).reshape(T, K)

    out = pl.pallas_call(
        _ffe_body,
        grid=(E,),
        in_specs=[
            pl.BlockSpec((T, K), lambda e: (0, 0)),
            pl.BlockSpec((T, K), lambda e: (0, 0)),
            pl.BlockSpec((T, D), lambda e: (0, 0)),
            pl.BlockSpec((1, BK, D), lambda e: (e, 0, 0)),
            pl.BlockSpec((1, D, BK), lambda e: (e, 0, 0)),
        ],
        out_specs=pl.BlockSpec((T, D), lambda e: (0, 0)),
        out_shape=jax.ShapeDtypeStruct((T, D), jnp.float32),
        compiler_params=pltpu.CompilerParams(
            dimension_semantics=("arbitrary",)),
    )(ens2, w2, x2, kernels_0, kernels_1)

    return out.reshape(B, S, D)


# speedup vs baseline: 11.1346x; 1.7342x over previous
import jax
import jax.numpy as jnp
from jax import lax
from jax.experimental import pallas as pl
from jax.experimental.pallas import tpu as pltpu


def _ffe_body(ens_ref, w_ref, x_ref, k0_ref, k1_ref, o_ref):
    T, K = ens_ref.shape
    E, D, BK = k1_ref.shape
    H = E * BK

    iota_e = lax.broadcasted_iota(jnp.int32, (1, E), 1)
    c = jnp.zeros((T, E), jnp.float32)
    for k in range(K):
        c = c + jnp.where(ens_ref[:, k:k + 1] == iota_e,
                          w_ref[:, k:k + 1], 0.0)

    blk = lax.broadcasted_iota(jnp.int32, (E, H), 1) // BK
    expand = jnp.where(lax.broadcasted_iota(jnp.int32, (E, H), 0) == blk,
                       1.0, 0.0)
    scale = jax.lax.dot_general(c, expand, (((1,), (0,)), ((), ())),
                                preferred_element_type=jnp.float32)

    h = jax.lax.dot_general(x_ref[...], k0_ref[...],
                            (((1,), (1,)), ((), ())),
                            preferred_element_type=jnp.float32)
    h = jnp.maximum(h, 0.0) * scale

    k1t = jnp.transpose(k1_ref[...], (0, 2, 1)).reshape(H, D)
    o_ref[...] = jax.lax.dot_general(h, k1t, (((1,), (0,)), ((), ())),
                                     preferred_element_type=jnp.float32)


def kernel(x, weights, ensembles, kernels_0, kernels_1):
    B, S, D = x.shape
    E, BK, _ = kernels_0.shape
    _, K, _ = weights.shape
    T = B * S

    x2 = x.reshape(T, D)
    ens2 = ensembles.transpose(0, 2, 1).reshape(T, K).astype(jnp.int32)
    w2 = weights.transpose(0, 2, 1).reshape(T, K)
    k0r = kernels_0.reshape(E * BK, D)

    out = pl.pallas_call(
        _ffe_body,
        out_shape=jax.ShapeDtypeStruct((T, D), jnp.float32),
    )(ens2, w2, x2, k0r, kernels_1)

    return out.reshape(B, S, D)
